# baseline (speedup 1.0000x reference)
import jax
import jax.numpy as jnp
from jax.experimental import pallas as pl
from jax.experimental.pallas import tpu as pltpu

B = 2
S = 77
K = 49408
N = 768
KB = 2048
NSTREAMS = 2
SLAB = NSTREAMS * KB
NSTEPS = -(-K // SLAB)
NBLK = K // KB


def _body(a_ref, b0_ref, b1_ref, p_ref, o_ref):
    k = pl.program_id(0)
    b_refs = (b0_ref, b1_ref)

    def full_slab():
        acc = jnp.zeros((S * B, N), jnp.float32)
        for j in range(NSTREAMS):
            a = a_ref[:, :, j * KB:(j + 1) * KB].reshape(S * B, KB)
            acc += jnp.dot(a.astype(jnp.bfloat16),
                           b_refs[j][...].astype(jnp.bfloat16),
                           preferred_element_type=jnp.float32)
        return acc

    def tail_slab():
        valid = K - (NSTEPS - 1) * SLAB
        a = a_ref[:, :, :KB].reshape(S * B, KB)
        a = jnp.where(
            jax.lax.broadcasted_iota(jnp.int32, a.shape, 1) < valid, a, 0.0)
        bm = b0_ref[...]
        bm = jnp.where(
            jax.lax.broadcasted_iota(jnp.int32, bm.shape, 0) < valid, bm, 0.0)
        return jnp.dot(a.astype(jnp.bfloat16), bm.astype(jnp.bfloat16),
                       preferred_element_type=jnp.float32)

    partial = jax.lax.cond(k == NSTEPS - 1, tail_slab, full_slab)

    @pl.when(k == 0)
    def _init():
        p = jnp.broadcast_to(p_ref[...][:, None, :], (S, B, N))
        o_ref[...] = partial.reshape(S, B, N) + p

    @pl.when(k > 0)
    def _acc():
        o_ref[...] += partial.reshape(S, B, N)


def _b_spec(j):
    return pl.BlockSpec(
        (KB, N), lambda k, j=j: (jnp.minimum(NSTREAMS * k + j, NBLK), 0))


@jax.jit
def kernel(input_ids, token_weight, position_weight):
    a_t = jnp.transpose(input_ids, (1, 0, 2))
    out_t = pl.pallas_call(
        _body,
        grid=(NSTEPS,),
        in_specs=[
            pl.BlockSpec((S, B, SLAB), lambda k: (0, 0, k)),
            _b_spec(0), _b_spec(1),
            pl.BlockSpec((S, N), lambda k: (0, 0)),
        ],
        out_specs=pl.BlockSpec((S, B, N), lambda k: (0, 0, 0)),
        out_shape=jax.ShapeDtypeStruct((S, B, N), jnp.float32),
        compiler_params=pltpu.CompilerParams(
            dimension_semantics=("arbitrary",)),
    )(a_t, token_weight, token_weight, position_weight)
    return jnp.transpose(out_t, (1, 0, 2))


# speedup vs baseline: 1.0387x; 1.0387x over previous
import jax
import jax.numpy as jnp
from jax.experimental import pallas as pl
from jax.experimental.pallas import tpu as pltpu

B = 2
S = 77
K = 49408
N = 768
KB = 2048
NSTEPS = -(-K // KB)


def _body(a_ref, b_ref, p_ref, o_ref):
    k = pl.program_id(0)

    def full_dot():
        a = a_ref[...].reshape(S * B, KB)
        return jnp.dot(a.astype(jnp.bfloat16),
                       b_ref[...].astype(jnp.bfloat16),
                       preferred_element_type=jnp.float32)

    def tail_dot():
        valid = K - (NSTEPS - 1) * KB
        a = a_ref[...].reshape(S * B, KB)
        a = jnp.where(
            jax.lax.broadcasted_iota(jnp.int32, a.shape, 1) < valid, a, 0.0)
        bm = b_ref[...]
        bm = jnp.where(
            jax.lax.broadcasted_iota(jnp.int32, bm.shape, 0) < valid, bm, 0.0)
        return jnp.dot(a.astype(jnp.bfloat16), bm.astype(jnp.bfloat16),
                       preferred_element_type=jnp.float32)

    partial = jax.lax.cond(k == NSTEPS - 1, tail_dot, full_dot)

    @pl.when(k == 0)
    def _init():
        p = jnp.broadcast_to(p_ref[...][:, None, :], (S, B, N))
        o_ref[...] = partial.reshape(S, B, N) + p

    @pl.when(k > 0)
    def _acc():
        o_ref[...] += partial.reshape(S, B, N)


@jax.jit
def kernel(input_ids, token_weight, position_weight):
    a_t = jnp.transpose(input_ids, (1, 0, 2))
    out_t = pl.pallas_call(
        _body,
        grid=(NSTEPS,),
        in_specs=[
            pl.BlockSpec((S, B, KB), lambda k: (0, 0, k)),
            pl.BlockSpec((KB, N), lambda k: (k, 0)),
            pl.BlockSpec((S, N), lambda k: (0, 0)),
        ],
        out_specs=pl.BlockSpec((S, B, N), lambda k: (0, 0, 0)),
        out_shape=jax.ShapeDtypeStruct((S, B, N), jnp.float32),
        compiler_params=pltpu.CompilerParams(
            dimension_semantics=("arbitrary",)),
    )(a_t, token_weight, position_weight)
    return jnp.transpose(out_t, (1, 0, 2))
